# baseline (speedup 1.0000x reference)
import functools
import math

import jax
import jax.numpy as jnp
from jax import lax
from jax.experimental import pallas as pl
from jax.experimental.pallas import tpu as pltpu
from jax.experimental.pallas import tpu_sc as plsc

D_MODEL = 64
SCALE = math.sqrt(D_MODEL)

NC = 2
NS = 16
NW = NC * NS
L = 16

NBUF = 4
LEAD = 2


def _make_emb_kernel(B0, B1):
    bw = B0 // NW
    bc = 4
    cb = bc * B1
    n_chunks = bw // bc
    row = B1 * D_MODEL
    mesh = plsc.VectorSubcoreMesh(core_axis_name="c", subcore_axis_name="s")

    @functools.partial(
        pl.kernel,
        mesh=mesh,
        compiler_params=pltpu.CompilerParams(
            use_tc_tiling_on_sc=False, needs_layout_passes=False
        ),
        out_type=jax.ShapeDtypeStruct((B0, row), jnp.float32),
        scratch_types=[
            pltpu.VMEM((B1, bw), jnp.int32),
            pltpu.VMEM((n_chunks, cb), jnp.int32),
            pltpu.VMEM((NBUF, cb, D_MODEL), jnp.float32),
            pltpu.VMEM((NBUF, bc, row), jnp.float32),
        ]
        + [pltpu.SemaphoreType.DMA] * (2 * NBUF),
    )
    def emb(xt_hbm, table_hbm, out_hbm, sx_v, idx_v, rows_v, st_v, *sems):
        gsems = sems[:NBUF]
        ssems = sems[NBUF:]
        wid = lax.axis_index("s") * NC + lax.axis_index("c")
        b0 = wid * bw
        pltpu.sync_copy(xt_hbm.at[:, pl.ds(b0, bw)], sx_v)

        iota16 = lax.broadcasted_iota(jnp.int32, (L,), 0)

        @plsc.parallel_loop(0, n_chunks, 1, unroll=2)
        def _(g):
            jg = bc * g
            for c16 in range(cb // L):
                p = c16 * L + iota16
                tv = lax.rem(p, B1)
                bv = lax.div(p, B1) + jg
                idx_v[g, pl.ds(c16 * L, L)] = plsc.load_gather(sx_v, [tv, bv])

        def start_gather(g, slot):
            pltpu.make_async_copy(
                table_hbm.at[idx_v.at[g]], rows_v.at[slot], gsems[slot]
            ).start()

        def wait_gather(g, slot):
            pltpu.make_async_copy(
                table_hbm.at[idx_v.at[g]], rows_v.at[slot], gsems[slot]
            ).wait()

        def start_scatter(g, slot):
            pltpu.make_async_copy(
                st_v.at[slot], out_hbm.at[pl.ds(b0 + g * bc, bc)], ssems[slot]
            ).start()

        def wait_scatter(slot):
            pltpu.make_async_copy(
                st_v.at[slot], out_hbm.at[pl.ds(b0, bc)], ssems[slot]
            ).wait()

        def scale_rearrange(slot):
            rv = rows_v.at[slot]
            sv = st_v.at[slot]

            @plsc.parallel_loop(0, B1, 1, unroll=2)
            def _(t):
                for bi in range(bc):
                    for c in range(D_MODEL // L):
                        sv[bi, pl.ds(t * D_MODEL + c * L, L)] = (
                            rv[bi * B1 + t, pl.ds(c * L, L)] * SCALE
                        )

        def process(g, slot):
            wait_gather(g, slot)
            scale_rearrange(slot)
            start_scatter(g, slot)

        start_gather(0, 0)
        start_gather(1, 1)
        process(0, 0)
        start_gather(2, 2)
        process(1, 1)
        start_gather(3, 3)

        def outer(go, carry):
            g0 = 2 + go * NBUF
            for bi in range(NBUF):
                g = g0 + bi
                slot = (2 + bi) % NBUF
                nslot = (slot + LEAD) % NBUF
                process(g, slot)
                wait_scatter(nslot)
                start_gather(g + LEAD, nslot)
            return carry

        lax.fori_loop(0, (n_chunks - NBUF) // NBUF, outer, 0)

        process(n_chunks - 2, (n_chunks - 2) % NBUF)
        process(n_chunks - 1, (n_chunks - 1) % NBUF)
        for s in range(NBUF):
            wait_scatter(s)

    return emb


def kernel(x, table):
    B0, B1 = x.shape
    assert B0 % NW == 0 and (B0 // NW) % 4 == 0
    xt = x.T.astype(jnp.int32)
    emb = _make_emb_kernel(B0, B1)
    out = emb(xt, table)
    return out.reshape(B0, B1, D_MODEL)
